# baseline (speedup 1.0000x reference)
import functools

import jax
import jax.numpy as jnp
from jax import lax
from jax.experimental import pallas as pl
from jax.experimental.pallas import tpu as pltpu
from jax.experimental.pallas import tpu_sc as plsc

_ALPHA = 0.01
_NEG_INF = -1e30
_BM = 256
_N = 2048
_D = 1024
_E = 8
_I = 512
_MR = _N * 2 + _E * _BM
_G = _MR // _BM
_NW = 32


def _router_body(x_ref, gwt_ref, pos0_ref, pos1_ref, w1b_ref, w2b_ref,
                 meta_ref, aux_ref):
    x = x_ref[...]
    gwt = gwt_ref[...]
    n = x.shape[0]
    logits = lax.dot_general(
        x, gwt, (((1,), (0,)), ((), ())),
        preferred_element_type=jnp.float32,
    )
    cols = lax.broadcasted_iota(jnp.int32, logits.shape, 1)
    valid_col = cols < _E
    masked = jnp.where(valid_col, logits, _NEG_INF)
    m = jnp.max(masked, axis=-1, keepdims=True)
    p = jnp.where(valid_col, jnp.exp(masked - m), 0.0)
    z = jnp.sum(p, axis=-1, keepdims=True)
    scores = p / z
    i1 = jnp.min(jnp.where(masked == m, cols, 127), axis=-1, keepdims=True)
    masked2 = jnp.where(cols == i1, _NEG_INF, masked)
    m2 = jnp.max(masked2, axis=-1, keepdims=True)
    i2 = jnp.min(jnp.where(masked2 == m2, cols, 127), axis=-1, keepdims=True)
    e1 = jnp.exp(m - m)
    e2 = jnp.exp(m2 - m)
    w1b_ref[...] = jnp.broadcast_to(e1 / (e1 + e2), (n, 16))
    w2b_ref[...] = jnp.broadcast_to(e2 / (e1 + e2), (n, 16))
    picks = (cols == i1).astype(jnp.float32) + (cols == i2).astype(jnp.float32)
    counts_tok = jnp.sum(picks, axis=0)
    sum_scores = jnp.sum(scores, axis=0)
    aux_ref[0, 0] = _ALPHA * _E * jnp.sum(
        (counts_tok / (n * 2)) * (sum_scores / n))

    i_slot = jnp.concatenate([i1, i2], axis=0)
    cols_a = lax.broadcasted_iota(jnp.int32, (2 * n, 128), 1)
    a = (cols_a == i_slot).astype(jnp.float32)
    c = 512
    ri = lax.broadcasted_iota(jnp.int32, (c, c), 0)
    ci = lax.broadcasted_iota(jnp.int32, (c, c), 1)
    l_strict = (ri > ci).astype(jnp.float32)
    running = jnp.zeros((1, 128), jnp.float32)
    p_blocks = []
    for cb in range(2 * n // c):
        blk = a[cb * c:(cb + 1) * c]
        p_in = lax.dot_general(l_strict, blk, (((1,), (0,)), ((), ())),
                               preferred_element_type=jnp.float32)
        p_blocks.append(p_in + running)
        running = running + jnp.sum(blk, axis=0, keepdims=True)
    prefix = jnp.concatenate(p_blocks, axis=0)
    counts = running
    pc = jnp.ceil(counts / _BM) * _BM
    ri8 = lax.broadcasted_iota(jnp.int32, (128, 128), 0)
    ci8 = lax.broadcasted_iota(jnp.int32, (128, 128), 1)
    u_strict = (ri8 < ci8).astype(jnp.float32)
    pad_off = lax.dot_general(pc, u_strict, (((1,), (0,)), ((), ())),
                              preferred_element_type=jnp.float32)
    cum = pad_off + pc
    r_tot = jnp.sum(jnp.where(cols[:1] == _E, pad_off, 0.0),
                    axis=1, keepdims=True)
    pos = jnp.sum(jnp.where(cols_a == i_slot, pad_off + prefix, 0.0),
                  axis=1, keepdims=True)
    pos_i = pos.astype(jnp.int32)
    pos0_ref[...] = pos_i[:n]
    pos1_ref[...] = pos_i[n:]
    s_lane = lax.broadcasted_iota(jnp.int32, (1, 128), 1)
    startf = (s_lane * _BM).astype(jnp.float32)
    eid = jnp.zeros((1, 128), jnp.float32)
    eid_last = jnp.zeros((1, 1), jnp.float32)
    for e in range(_E):
        ce = jnp.sum(jnp.where(cols[:1] == e, cum, 0.0), axis=1, keepdims=True)
        eid = eid + (startf >= ce).astype(jnp.float32)
        eid_last = eid_last + (r_tot - _BM >= ce).astype(jnp.float32)
    blk_valid = startf < r_tot
    eid = jnp.where(blk_valid, eid, eid_last)
    bid = jnp.where(blk_valid, s_lane,
                    (r_tot / _BM).astype(jnp.int32) - 1)
    meta = jnp.concatenate([
        eid.astype(jnp.int32),
        bid.astype(jnp.int32),
        blk_valid.astype(jnp.int32),
        jnp.broadcast_to(r_tot.astype(jnp.int32), (1, 128)),
        jnp.zeros((4, 128), jnp.int32),
    ], axis=0)
    meta_ref[...] = meta


def _sc_scatter_body(flat_hbm, pos0_hbm, pos1_hbm, xpad_hbm,
                     rows_a, rows_b, idx0a_v, idx1a_v, idx0b_v, idx1b_v,
                     lsem_a, lsem_b, ssem):
    wid = lax.axis_index("s") * 2 + lax.axis_index("c")
    tpw = _N // _NW
    sub = tpw // 2
    b0 = wid * tpw
    b1 = b0 + sub
    la = pltpu.async_copy(flat_hbm.at[pl.ds(b0, sub)], rows_a, lsem_a)
    lb = pltpu.async_copy(flat_hbm.at[pl.ds(b1, sub)], rows_b, lsem_b)
    pltpu.sync_copy(pos0_hbm.at[pl.ds(b0, sub)], idx0a_v)
    pltpu.sync_copy(pos1_hbm.at[pl.ds(b0, sub)], idx1a_v)
    pltpu.sync_copy(pos0_hbm.at[pl.ds(b1, sub)], idx0b_v)
    pltpu.sync_copy(pos1_hbm.at[pl.ds(b1, sub)], idx1b_v)
    la.wait()
    s0a = pltpu.async_copy(rows_a, xpad_hbm.at[idx0a_v], ssem)
    s1a = pltpu.async_copy(rows_a, xpad_hbm.at[idx1a_v], ssem)
    lb.wait()
    s0b = pltpu.async_copy(rows_b, xpad_hbm.at[idx0b_v], ssem)
    s1b = pltpu.async_copy(rows_b, xpad_hbm.at[idx1b_v], ssem)
    s0a.wait()
    s1a.wait()
    s0b.wait()
    s1b.wait()


def _shared_body(x_ref, sg_ref, su_ref, sd_ref, y_ref):
    x = x_ref[...].astype(jnp.bfloat16)
    sg = sg_ref[0].astype(jnp.bfloat16)
    su = su_ref[0].astype(jnp.bfloat16)
    sd = sd_ref[0].astype(jnp.bfloat16)
    g = lax.dot_general(x, sg, (((1,), (1,)), ((), ())),
                        preferred_element_type=jnp.float32)
    u = lax.dot_general(x, su, (((1,), (1,)), ((), ())),
                        preferred_element_type=jnp.float32)
    act = (g * jax.nn.sigmoid(g) * u).astype(jnp.bfloat16)
    y = lax.dot_general(act, sd, (((1,), (1,)), ((), ())),
                        preferred_element_type=jnp.float32)
    y_ref[...] = y.astype(jnp.bfloat16)


def _gmm_body(eid_ref, bid_ref, valid_ref, x_ref, wg_ref, wu_ref, wd_ref,
              y_ref):
    s = pl.program_id(0)
    h = pl.program_id(1)

    @pl.when(valid_ref[s] == 1)
    def _():
        x = x_ref[...].astype(jnp.bfloat16)
        wg = wg_ref[0].astype(jnp.bfloat16)
        wu = wu_ref[0].astype(jnp.bfloat16)
        wd = wd_ref[0].astype(jnp.bfloat16)
        g = lax.dot_general(x, wg, (((1,), (1,)), ((), ())),
                            preferred_element_type=jnp.float32)
        u = lax.dot_general(x, wu, (((1,), (1,)), ((), ())),
                            preferred_element_type=jnp.float32)
        act = (g * jax.nn.sigmoid(g) * u).astype(jnp.bfloat16)
        contrib = lax.dot_general(act, wd, (((1,), (1,)), ((), ())),
                                  preferred_element_type=jnp.float32)

        @pl.when(h == 0)
        def _init():
            y_ref[...] = contrib

        @pl.when(h != 0)
        def _acc():
            y_ref[...] += contrib


def _sc_gather_body(ypad_hbm, pos0_hbm, pos1_hbm, y0g_hbm, y1g_hbm,
                    buf_a, buf_b, buf_c,
                    i0a_v, i1a_v, i0b_v, i1b_v,
                    sem_a, sem_b, sem_c, wsem_a, wsem_b, wsem_c):
    wid = lax.axis_index("s") * 2 + lax.axis_index("c")
    tpw = _N // _NW
    sub = tpw // 2
    b0 = wid * tpw
    b1 = b0 + sub
    pltpu.sync_copy(pos0_hbm.at[pl.ds(b0, sub)], i0a_v)
    pltpu.sync_copy(pos1_hbm.at[pl.ds(b0, sub)], i1a_v)
    pltpu.sync_copy(pos0_hbm.at[pl.ds(b1, sub)], i0b_v)
    pltpu.sync_copy(pos1_hbm.at[pl.ds(b1, sub)], i1b_v)
    g0 = pltpu.async_copy(ypad_hbm.at[i0a_v], buf_a, sem_a)
    g1 = pltpu.async_copy(ypad_hbm.at[i1a_v], buf_b, sem_b)
    g0.wait()
    w_a = pltpu.async_copy(buf_a, y0g_hbm.at[pl.ds(b0, sub)], wsem_a)
    g2 = pltpu.async_copy(ypad_hbm.at[i0b_v], buf_c, sem_c)
    g1.wait()
    w_b = pltpu.async_copy(buf_b, y1g_hbm.at[pl.ds(b0, sub)], wsem_b)
    w_a.wait()
    g3 = pltpu.async_copy(ypad_hbm.at[i1b_v], buf_a, sem_a)
    g2.wait()
    w_c = pltpu.async_copy(buf_c, y0g_hbm.at[pl.ds(b1, sub)], wsem_c)
    g3.wait()
    w_a2 = pltpu.async_copy(buf_a, y1g_hbm.at[pl.ds(b1, sub)], wsem_a)
    w_b.wait()
    w_c.wait()
    w_a2.wait()


def _add_body(y0_ref, y1_ref, ysh_ref, w1_ref, w2_ref, out_ref):
    w1 = w1_ref[...][:, :1]
    w2 = w2_ref[...][:, :1]
    ysh = ysh_ref[...].astype(jnp.float32)
    out_ref[...] = w1 * y0_ref[...] + w2 * y1_ref[...] + ysh


def kernel(hidden_states, gate_w, Wg, Wu, Wd, Sg, Su, Sd):
    b, t, d = hidden_states.shape
    n = b * t
    flat = hidden_states.reshape(n, d)
    gwt = jnp.zeros((d, 128), jnp.float32).at[:, :_E].set(gate_w.T)

    pos0, pos1, w1b, w2b, meta, aux = pl.pallas_call(
        _router_body,
        out_shape=(
            jax.ShapeDtypeStruct((n, 1), jnp.int32),
            jax.ShapeDtypeStruct((n, 1), jnp.int32),
            jax.ShapeDtypeStruct((n, 16), jnp.float32),
            jax.ShapeDtypeStruct((n, 16), jnp.float32),
            jax.ShapeDtypeStruct((8, 128), jnp.int32),
            jax.ShapeDtypeStruct((1, 1), jnp.float32),
        ),
        in_specs=[
            pl.BlockSpec((n, d), lambda: (0, 0)),
            pl.BlockSpec((d, 128), lambda: (0, 0)),
        ],
        out_specs=(
            pl.BlockSpec((n, 1), lambda: (0, 0)),
            pl.BlockSpec((n, 1), lambda: (0, 0)),
            pl.BlockSpec((n, 16), lambda: (0, 0)),
            pl.BlockSpec((n, 16), lambda: (0, 0)),
            pl.BlockSpec((8, 128), lambda: (0, 0)),
            pl.BlockSpec(memory_space=pltpu.SMEM),
        ),
    )(flat, gwt)

    pos0_f = pos0.reshape(n)
    pos1_f = pos1.reshape(n)
    eid = meta[0, :_G]
    bid = meta[1, :_G]
    blk_valid = meta[2, :_G]

    mesh = plsc.VectorSubcoreMesh(core_axis_name="c", subcore_axis_name="s")
    sc_scatter = functools.partial(
        pl.kernel, mesh=mesh,
        out_type=jax.ShapeDtypeStruct((_MR, d), jnp.float32),
        scratch_types=[
            pltpu.VMEM((32, d), jnp.float32),
            pltpu.VMEM((32, d), jnp.float32),
            pltpu.VMEM((32,), jnp.int32),
            pltpu.VMEM((32,), jnp.int32),
            pltpu.VMEM((32,), jnp.int32),
            pltpu.VMEM((32,), jnp.int32),
            pltpu.SemaphoreType.DMA,
            pltpu.SemaphoreType.DMA,
            pltpu.SemaphoreType.DMA,
        ],
    )(_sc_scatter_body)
    x_pad = sc_scatter(flat, pos0_f, pos1_f)

    ysh = pl.pallas_call(
        _shared_body,
        grid=(4,),
        out_shape=jax.ShapeDtypeStruct((n, d), jnp.bfloat16),
        in_specs=[
            pl.BlockSpec((n // 4, d), lambda s: (s, 0)),
            pl.BlockSpec((1, _I, d), lambda s: (0, 0, 0)),
            pl.BlockSpec((1, _I, d), lambda s: (0, 0, 0)),
            pl.BlockSpec((1, d, _I), lambda s: (0, 0, 0)),
        ],
        out_specs=pl.BlockSpec((n // 4, d), lambda s: (s, 0)),
    )(flat, Sg, Su, Sd)

    grid_spec = pltpu.PrefetchScalarGridSpec(
        num_scalar_prefetch=3,
        grid=(_G, 2),
        in_specs=[
            pl.BlockSpec((_BM, d), lambda s, h, eid, bid, vld: (bid[s], 0)),
            pl.BlockSpec((1, _I // 2, d),
                         lambda s, h, eid, bid, vld: (eid[s], h, 0)),
            pl.BlockSpec((1, _I // 2, d),
                         lambda s, h, eid, bid, vld: (eid[s], h, 0)),
            pl.BlockSpec((1, d, _I // 2),
                         lambda s, h, eid, bid, vld: (eid[s], 0, h)),
        ],
        out_specs=pl.BlockSpec((_BM, d),
                               lambda s, h, eid, bid, vld: (bid[s], 0)),
    )
    y_pad = pl.pallas_call(
        _gmm_body,
        grid_spec=grid_spec,
        out_shape=jax.ShapeDtypeStruct((_MR, d), jnp.float32),
    )(eid, bid, blk_valid, x_pad, Wg, Wu, Wd)

    sc_gather = functools.partial(
        pl.kernel, mesh=mesh,
        out_type=(
            jax.ShapeDtypeStruct((n, d), jnp.float32),
            jax.ShapeDtypeStruct((n, d), jnp.float32),
        ),
        scratch_types=[
            pltpu.VMEM((32, d), jnp.float32),
            pltpu.VMEM((32, d), jnp.float32),
            pltpu.VMEM((32, d), jnp.float32),
            pltpu.VMEM((32,), jnp.int32),
            pltpu.VMEM((32,), jnp.int32),
            pltpu.VMEM((32,), jnp.int32),
            pltpu.VMEM((32,), jnp.int32),
            pltpu.SemaphoreType.DMA,
            pltpu.SemaphoreType.DMA,
            pltpu.SemaphoreType.DMA,
            pltpu.SemaphoreType.DMA,
            pltpu.SemaphoreType.DMA,
            pltpu.SemaphoreType.DMA,
        ],
    )(_sc_gather_body)
    y0g, y1g = sc_gather(y_pad, pos0_f, pos1_f)

    out_flat = pl.pallas_call(
        _add_body,
        grid=(4,),
        out_shape=jax.ShapeDtypeStruct((n, d), jnp.float32),
        in_specs=[
            pl.BlockSpec((n // 4, d), lambda s: (s, 0)),
            pl.BlockSpec((n // 4, d), lambda s: (s, 0)),
            pl.BlockSpec((n // 4, d), lambda s: (s, 0)),
            pl.BlockSpec((n // 4, 16), lambda s: (s, 0)),
            pl.BlockSpec((n // 4, 16), lambda s: (s, 0)),
        ],
        out_specs=pl.BlockSpec((n // 4, d), lambda s: (s, 0)),
    )(y0g, y1g, ysh, w1b, w2b)

    return out_flat.reshape(b, t, d), aux[0, 0]


# speedup vs baseline: 1.2542x; 1.2542x over previous
import functools

import jax
import jax.numpy as jnp
from jax import lax
from jax.experimental import pallas as pl
from jax.experimental.pallas import tpu as pltpu
from jax.experimental.pallas import tpu_sc as plsc

_ALPHA = 0.01
_NEG_INF = -1e30
_BM = 256
_N = 2048
_D = 1024
_E = 8
_I = 512
_MR = _N * 2 + _E * _BM
_G = _MR // _BM
_NW = 32


def _router_body(x_ref, gwt_ref, pos_ref, w1b_ref, w2b_ref,
                 meta_ref, aux_ref):
    x = x_ref[...]
    gwt = gwt_ref[...]
    n = x.shape[0]
    logits = lax.dot_general(
        x, gwt, (((1,), (0,)), ((), ())),
        preferred_element_type=jnp.float32,
    )
    cols = lax.broadcasted_iota(jnp.int32, logits.shape, 1)
    valid_col = cols < _E
    masked = jnp.where(valid_col, logits, _NEG_INF)
    m = jnp.max(masked, axis=-1, keepdims=True)
    p = jnp.where(valid_col, jnp.exp(masked - m), 0.0)
    z = jnp.sum(p, axis=-1, keepdims=True)
    scores = p / z
    i1 = jnp.min(jnp.where(masked == m, cols, 127), axis=-1, keepdims=True)
    masked2 = jnp.where(cols == i1, _NEG_INF, masked)
    m2 = jnp.max(masked2, axis=-1, keepdims=True)
    i2 = jnp.min(jnp.where(masked2 == m2, cols, 127), axis=-1, keepdims=True)
    e1 = jnp.exp(m - m)
    e2 = jnp.exp(m2 - m)
    w1b_ref[...] = jnp.broadcast_to(e1 / (e1 + e2), (n, 16))
    w2b_ref[...] = jnp.broadcast_to(e2 / (e1 + e2), (n, 16))
    picks = (cols == i1).astype(jnp.float32) + (cols == i2).astype(jnp.float32)
    counts_tok = jnp.sum(picks, axis=0)
    sum_scores = jnp.sum(scores, axis=0)
    aux_ref[0, 0] = _ALPHA * _E * jnp.sum(
        (counts_tok / (n * 2)) * (sum_scores / n))

    i_slot = jnp.concatenate([i1, i2], axis=0)
    cols_a = lax.broadcasted_iota(jnp.int32, (2 * n, 128), 1)
    a = (cols_a == i_slot).astype(jnp.float32)
    c = 512
    ri = lax.broadcasted_iota(jnp.int32, (c, c), 0)
    ci = lax.broadcasted_iota(jnp.int32, (c, c), 1)
    l_strict = (ri > ci).astype(jnp.float32)
    running = jnp.zeros((1, 128), jnp.float32)
    p_blocks = []
    for cb in range(2 * n // c):
        blk = a[cb * c:(cb + 1) * c]
        p_in = lax.dot_general(l_strict, blk, (((1,), (0,)), ((), ())),
                               preferred_element_type=jnp.float32)
        p_blocks.append(p_in + running)
        running = running + jnp.sum(blk, axis=0, keepdims=True)
    prefix = jnp.concatenate(p_blocks, axis=0)
    counts = running
    pc = jnp.ceil(counts / _BM) * _BM
    ri8 = lax.broadcasted_iota(jnp.int32, (128, 128), 0)
    ci8 = lax.broadcasted_iota(jnp.int32, (128, 128), 1)
    u_strict = (ri8 < ci8).astype(jnp.float32)
    pad_off = lax.dot_general(pc, u_strict, (((1,), (0,)), ((), ())),
                              preferred_element_type=jnp.float32)
    cum = pad_off + pc
    r_tot = jnp.sum(jnp.where(cols[:1] == _E, pad_off, 0.0),
                    axis=1, keepdims=True)
    pos = jnp.sum(jnp.where(cols_a == i_slot, pad_off + prefix, 0.0),
                  axis=1, keepdims=True)
    pos_ref[...] = pos.astype(jnp.int32)
    s_lane = lax.broadcasted_iota(jnp.int32, (1, 128), 1)
    startf = (s_lane * _BM).astype(jnp.float32)
    eid = jnp.zeros((1, 128), jnp.float32)
    eid_last = jnp.zeros((1, 1), jnp.float32)
    for e in range(_E):
        ce = jnp.sum(jnp.where(cols[:1] == e, cum, 0.0), axis=1, keepdims=True)
        eid = eid + (startf >= ce).astype(jnp.float32)
        eid_last = eid_last + (r_tot - _BM >= ce).astype(jnp.float32)
    blk_valid = startf < r_tot
    eid = jnp.where(blk_valid, eid, eid_last)
    bid = jnp.where(blk_valid, s_lane,
                    (r_tot / _BM).astype(jnp.int32) - 1)
    meta = jnp.concatenate([
        eid.astype(jnp.int32),
        bid.astype(jnp.int32),
        blk_valid.astype(jnp.int32),
        jnp.broadcast_to(r_tot.astype(jnp.int32), (1, 128)),
        jnp.zeros((4, 128), jnp.int32),
    ], axis=0)
    meta_ref[...] = meta


def _sc_scatter_body(flat_hbm, pos_hbm, xpad_hbm,
                     rows_a, rows_b, idx0a_v, idx1a_v, idx0b_v, idx1b_v,
                     lsem_a, lsem_b, ssem):
    wid = lax.axis_index("s") * 2 + lax.axis_index("c")
    tpw = _N // _NW
    sub = tpw // 2
    b0 = wid * tpw
    b1 = b0 + sub
    la = pltpu.async_copy(flat_hbm.at[pl.ds(b0, sub)], rows_a, lsem_a)
    lb = pltpu.async_copy(flat_hbm.at[pl.ds(b1, sub)], rows_b, lsem_b)
    pltpu.sync_copy(pos_hbm.at[pl.ds(b0, sub)], idx0a_v)
    pltpu.sync_copy(pos_hbm.at[pl.ds(_N + b0, sub)], idx1a_v)
    pltpu.sync_copy(pos_hbm.at[pl.ds(b1, sub)], idx0b_v)
    pltpu.sync_copy(pos_hbm.at[pl.ds(_N + b1, sub)], idx1b_v)
    la.wait()
    s0a = pltpu.async_copy(rows_a, xpad_hbm.at[idx0a_v], ssem)
    s1a = pltpu.async_copy(rows_a, xpad_hbm.at[idx1a_v], ssem)
    lb.wait()
    s0b = pltpu.async_copy(rows_b, xpad_hbm.at[idx0b_v], ssem)
    s1b = pltpu.async_copy(rows_b, xpad_hbm.at[idx1b_v], ssem)
    s0a.wait()
    s1a.wait()
    s0b.wait()
    s1b.wait()


def _shared_body(x_ref, sg_ref, su_ref, sd_ref, y_ref):
    x = x_ref[...].astype(jnp.bfloat16)
    sg = sg_ref[0].astype(jnp.bfloat16)
    su = su_ref[0].astype(jnp.bfloat16)
    sd = sd_ref[0].astype(jnp.bfloat16)
    g = lax.dot_general(x, sg, (((1,), (1,)), ((), ())),
                        preferred_element_type=jnp.float32)
    u = lax.dot_general(x, su, (((1,), (1,)), ((), ())),
                        preferred_element_type=jnp.float32)
    act = (g * jax.nn.sigmoid(g) * u).astype(jnp.bfloat16)
    y = lax.dot_general(act, sd, (((1,), (1,)), ((), ())),
                        preferred_element_type=jnp.float32)
    y_ref[...] = y.astype(jnp.bfloat16)


def _gmm_body(eid_ref, bid_ref, valid_ref, x_ref, wg_ref, wu_ref, wd_ref,
              y_ref):
    s = pl.program_id(0)

    @pl.when(valid_ref[s] == 1)
    def _():
        x = x_ref[...].astype(jnp.bfloat16)
        wg = wg_ref[0].astype(jnp.bfloat16)
        wu = wu_ref[0].astype(jnp.bfloat16)
        wd = wd_ref[0].astype(jnp.bfloat16)
        g = lax.dot_general(x, wg, (((1,), (1,)), ((), ())),
                            preferred_element_type=jnp.float32)
        u = lax.dot_general(x, wu, (((1,), (1,)), ((), ())),
                            preferred_element_type=jnp.float32)
        act = (g * jax.nn.sigmoid(g) * u).astype(jnp.bfloat16)
        y_ref[...] = lax.dot_general(act, wd, (((1,), (1,)), ((), ())),
                                     preferred_element_type=jnp.float32)


def _sc_gather_body(ypad_hbm, pos_hbm, y0g_hbm, y1g_hbm,
                    buf_a, buf_b, buf_c,
                    i0a_v, i1a_v, i0b_v, i1b_v,
                    sem_a, sem_b, sem_c, wsem_a, wsem_b, wsem_c):
    wid = lax.axis_index("s") * 2 + lax.axis_index("c")
    tpw = _N // _NW
    sub = tpw // 2
    b0 = wid * tpw
    b1 = b0 + sub
    pltpu.sync_copy(pos_hbm.at[pl.ds(b0, sub)], i0a_v)
    pltpu.sync_copy(pos_hbm.at[pl.ds(_N + b0, sub)], i1a_v)
    pltpu.sync_copy(pos_hbm.at[pl.ds(b1, sub)], i0b_v)
    pltpu.sync_copy(pos_hbm.at[pl.ds(_N + b1, sub)], i1b_v)
    g0 = pltpu.async_copy(ypad_hbm.at[i0a_v], buf_a, sem_a)
    g1 = pltpu.async_copy(ypad_hbm.at[i1a_v], buf_b, sem_b)
    g0.wait()
    w_a = pltpu.async_copy(buf_a, y0g_hbm.at[pl.ds(b0, sub)], wsem_a)
    g2 = pltpu.async_copy(ypad_hbm.at[i0b_v], buf_c, sem_c)
    g1.wait()
    w_b = pltpu.async_copy(buf_b, y1g_hbm.at[pl.ds(b0, sub)], wsem_b)
    w_a.wait()
    g3 = pltpu.async_copy(ypad_hbm.at[i1b_v], buf_a, sem_a)
    g2.wait()
    w_c = pltpu.async_copy(buf_c, y0g_hbm.at[pl.ds(b1, sub)], wsem_c)
    g3.wait()
    w_a2 = pltpu.async_copy(buf_a, y1g_hbm.at[pl.ds(b1, sub)], wsem_a)
    w_b.wait()
    w_c.wait()
    w_a2.wait()


def _add_body(y0_ref, y1_ref, ysh_ref, w1_ref, w2_ref, out_ref):
    w1 = w1_ref[...][:, :1]
    w2 = w2_ref[...][:, :1]
    ysh = ysh_ref[...].astype(jnp.float32)
    out_ref[...] = w1 * y0_ref[...] + w2 * y1_ref[...] + ysh


def kernel(hidden_states, gate_w, Wg, Wu, Wd, Sg, Su, Sd):
    b, t, d = hidden_states.shape
    n = b * t
    flat = hidden_states.reshape(n, d)
    gwt = jnp.pad(gate_w.T, ((0, 0), (0, 128 - _E)))

    pos, w1b, w2b, meta, aux = pl.pallas_call(
        _router_body,
        out_shape=(
            jax.ShapeDtypeStruct((2 * n, 1), jnp.int32),
            jax.ShapeDtypeStruct((n, 16), jnp.float32),
            jax.ShapeDtypeStruct((n, 16), jnp.float32),
            jax.ShapeDtypeStruct((8, 128), jnp.int32),
            jax.ShapeDtypeStruct((1, 1), jnp.float32),
        ),
        in_specs=[
            pl.BlockSpec((n, d), lambda: (0, 0)),
            pl.BlockSpec((d, 128), lambda: (0, 0)),
        ],
        out_specs=(
            pl.BlockSpec((2 * n, 1), lambda: (0, 0)),
            pl.BlockSpec((n, 16), lambda: (0, 0)),
            pl.BlockSpec((n, 16), lambda: (0, 0)),
            pl.BlockSpec((8, 128), lambda: (0, 0)),
            pl.BlockSpec(memory_space=pltpu.SMEM),
        ),
    )(flat, gwt)

    pos_all = pos.reshape(2 * n)
    eid = meta[0, :_G]
    bid = meta[1, :_G]
    blk_valid = meta[2, :_G]

    mesh = plsc.VectorSubcoreMesh(core_axis_name="c", subcore_axis_name="s")
    sc_scatter = functools.partial(
        pl.kernel, mesh=mesh,
        out_type=jax.ShapeDtypeStruct((_MR, d), jnp.float32),
        scratch_types=[
            pltpu.VMEM((32, d), jnp.float32),
            pltpu.VMEM((32, d), jnp.float32),
            pltpu.VMEM((32,), jnp.int32),
            pltpu.VMEM((32,), jnp.int32),
            pltpu.VMEM((32,), jnp.int32),
            pltpu.VMEM((32,), jnp.int32),
            pltpu.SemaphoreType.DMA,
            pltpu.SemaphoreType.DMA,
            pltpu.SemaphoreType.DMA,
        ],
    )(_sc_scatter_body)
    x_pad = sc_scatter(flat, pos_all)

    ysh = pl.pallas_call(
        _shared_body,
        grid=(4,),
        out_shape=jax.ShapeDtypeStruct((n, d), jnp.bfloat16),
        in_specs=[
            pl.BlockSpec((n // 4, d), lambda s: (s, 0)),
            pl.BlockSpec((1, _I, d), lambda s: (0, 0, 0)),
            pl.BlockSpec((1, _I, d), lambda s: (0, 0, 0)),
            pl.BlockSpec((1, d, _I), lambda s: (0, 0, 0)),
        ],
        out_specs=pl.BlockSpec((n // 4, d), lambda s: (s, 0)),
    )(flat, Sg, Su, Sd)

    grid_spec = pltpu.PrefetchScalarGridSpec(
        num_scalar_prefetch=3,
        grid=(_G,),
        in_specs=[
            pl.BlockSpec((_BM, d), lambda s, eid, bid, vld: (bid[s], 0)),
            pl.BlockSpec((1, _I, d), lambda s, eid, bid, vld: (eid[s], 0, 0)),
            pl.BlockSpec((1, _I, d), lambda s, eid, bid, vld: (eid[s], 0, 0)),
            pl.BlockSpec((1, d, _I), lambda s, eid, bid, vld: (eid[s], 0, 0)),
        ],
        out_specs=pl.BlockSpec((_BM, d), lambda s, eid, bid, vld: (bid[s], 0)),
    )
    y_pad = pl.pallas_call(
        _gmm_body,
        grid_spec=grid_spec,
        out_shape=jax.ShapeDtypeStruct((_MR, d), jnp.float32),
    )(eid, bid, blk_valid, x_pad, Wg, Wu, Wd)

    sc_gather = functools.partial(
        pl.kernel, mesh=mesh,
        out_type=(
            jax.ShapeDtypeStruct((n, d), jnp.float32),
            jax.ShapeDtypeStruct((n, d), jnp.float32),
        ),
        scratch_types=[
            pltpu.VMEM((32, d), jnp.float32),
            pltpu.VMEM((32, d), jnp.float32),
            pltpu.VMEM((32, d), jnp.float32),
            pltpu.VMEM((32,), jnp.int32),
            pltpu.VMEM((32,), jnp.int32),
            pltpu.VMEM((32,), jnp.int32),
            pltpu.VMEM((32,), jnp.int32),
            pltpu.SemaphoreType.DMA,
            pltpu.SemaphoreType.DMA,
            pltpu.SemaphoreType.DMA,
            pltpu.SemaphoreType.DMA,
            pltpu.SemaphoreType.DMA,
            pltpu.SemaphoreType.DMA,
        ],
    )(_sc_gather_body)
    y0g, y1g = sc_gather(y_pad, pos_all)

    out_flat = pl.pallas_call(
        _add_body,
        grid=(4,),
        out_shape=jax.ShapeDtypeStruct((n, d), jnp.float32),
        in_specs=[
            pl.BlockSpec((n // 4, d), lambda s: (s, 0)),
            pl.BlockSpec((n // 4, d), lambda s: (s, 0)),
            pl.BlockSpec((n // 4, d), lambda s: (s, 0)),
            pl.BlockSpec((n // 4, 16), lambda s: (s, 0)),
            pl.BlockSpec((n // 4, 16), lambda s: (s, 0)),
        ],
        out_specs=pl.BlockSpec((n // 4, d), lambda s: (s, 0)),
    )(y0g, y1g, ysh, w1b, w2b)

    return out_flat.reshape(b, t, d), aux[0, 0]


# speedup vs baseline: 1.3612x; 1.0853x over previous
import functools

import jax
import jax.numpy as jnp
from jax import lax
from jax.experimental import pallas as pl
from jax.experimental.pallas import tpu as pltpu
from jax.experimental.pallas import tpu_sc as plsc

_ALPHA = 0.01
_NEG_INF = -1e30
_BM = 512
_N = 2048
_D = 1024
_E = 8
_I = 512
_MR = _N * 2 + _E * _BM
_G = _MR // _BM
_NW = 32


def _router_body(x_ref, gwt_ref, pos_ref, w1b_ref, w2b_ref,
                 meta_ref, aux_ref):
    x = x_ref[...]
    gwt = gwt_ref[...]
    n = x.shape[0]
    logits = lax.dot_general(
        x, gwt, (((1,), (0,)), ((), ())),
        preferred_element_type=jnp.float32,
    )
    cols = lax.broadcasted_iota(jnp.int32, logits.shape, 1)
    valid_col = cols < _E
    masked = jnp.where(valid_col, logits, _NEG_INF)
    m = jnp.max(masked, axis=-1, keepdims=True)
    p = jnp.where(valid_col, jnp.exp(masked - m), 0.0)
    z = jnp.sum(p, axis=-1, keepdims=True)
    scores = p / z
    i1 = jnp.min(jnp.where(masked == m, cols, 127), axis=-1, keepdims=True)
    masked2 = jnp.where(cols == i1, _NEG_INF, masked)
    m2 = jnp.max(masked2, axis=-1, keepdims=True)
    i2 = jnp.min(jnp.where(masked2 == m2, cols, 127), axis=-1, keepdims=True)
    e1 = jnp.exp(m - m)
    e2 = jnp.exp(m2 - m)
    w1b_ref[...] = jnp.broadcast_to(e1 / (e1 + e2), (n, 16))
    w2b_ref[...] = jnp.broadcast_to(e2 / (e1 + e2), (n, 16))
    picks = (cols == i1).astype(jnp.float32) + (cols == i2).astype(jnp.float32)
    counts_tok = jnp.sum(picks, axis=0)
    sum_scores = jnp.sum(scores, axis=0)
    aux_ref[0, 0] = _ALPHA * _E * jnp.sum(
        (counts_tok / (n * 2)) * (sum_scores / n))

    i_slot = jnp.concatenate([i1, i2], axis=0)
    cols_a = lax.broadcasted_iota(jnp.int32, (2 * n, 128), 1)
    a = (cols_a == i_slot).astype(jnp.float32)
    c = 512
    ri = lax.broadcasted_iota(jnp.int32, (c, c), 0)
    ci = lax.broadcasted_iota(jnp.int32, (c, c), 1)
    l_strict = (ri > ci).astype(jnp.float32)
    running = jnp.zeros((1, 128), jnp.float32)
    p_blocks = []
    for cb in range(2 * n // c):
        blk = a[cb * c:(cb + 1) * c]
        p_in = lax.dot_general(l_strict, blk, (((1,), (0,)), ((), ())),
                               preferred_element_type=jnp.float32)
        p_blocks.append(p_in + running)
        running = running + jnp.sum(blk, axis=0, keepdims=True)
    prefix = jnp.concatenate(p_blocks, axis=0)
    counts = running
    pc = jnp.ceil(counts / _BM) * _BM
    ri8 = lax.broadcasted_iota(jnp.int32, (128, 128), 0)
    ci8 = lax.broadcasted_iota(jnp.int32, (128, 128), 1)
    u_strict = (ri8 < ci8).astype(jnp.float32)
    pad_off = lax.dot_general(pc, u_strict, (((1,), (0,)), ((), ())),
                              preferred_element_type=jnp.float32)
    cum = pad_off + pc
    r_tot = jnp.sum(jnp.where(cols[:1] == _E, pad_off, 0.0),
                    axis=1, keepdims=True)
    pos = jnp.sum(jnp.where(cols_a == i_slot, pad_off + prefix, 0.0),
                  axis=1, keepdims=True)
    pos_ref[...] = pos.astype(jnp.int32)
    s_lane = lax.broadcasted_iota(jnp.int32, (1, 128), 1)
    startf = (s_lane * _BM).astype(jnp.float32)
    eid = jnp.zeros((1, 128), jnp.float32)
    eid_last = jnp.zeros((1, 1), jnp.float32)
    for e in range(_E):
        ce = jnp.sum(jnp.where(cols[:1] == e, cum, 0.0), axis=1, keepdims=True)
        eid = eid + (startf >= ce).astype(jnp.float32)
        eid_last = eid_last + (r_tot - _BM >= ce).astype(jnp.float32)
    blk_valid = startf < r_tot
    eid = jnp.where(blk_valid, eid, eid_last)
    bid = jnp.where(blk_valid, s_lane,
                    (r_tot / _BM).astype(jnp.int32) - 1)
    meta = jnp.concatenate([
        eid.astype(jnp.int32),
        bid.astype(jnp.int32),
        blk_valid.astype(jnp.int32),
        jnp.broadcast_to(r_tot.astype(jnp.int32), (1, 128)),
        jnp.zeros((4, 128), jnp.int32),
    ], axis=0)
    meta_ref[...] = meta


def _sc_scatter_body(flat_hbm, pos_hbm, xpad_hbm,
                     rows_a, rows_b, idx0a_v, idx1a_v, idx0b_v, idx1b_v,
                     lsem_a, lsem_b, ssem):
    wid = lax.axis_index("s") * 2 + lax.axis_index("c")
    tpw = _N // _NW
    sub = tpw // 2
    b0 = wid * tpw
    b1 = b0 + sub
    la = pltpu.async_copy(flat_hbm.at[pl.ds(b0, sub)], rows_a, lsem_a)
    lb = pltpu.async_copy(flat_hbm.at[pl.ds(b1, sub)], rows_b, lsem_b)
    pltpu.sync_copy(pos_hbm.at[pl.ds(b0, sub)], idx0a_v)
    pltpu.sync_copy(pos_hbm.at[pl.ds(_N + b0, sub)], idx1a_v)
    pltpu.sync_copy(pos_hbm.at[pl.ds(b1, sub)], idx0b_v)
    pltpu.sync_copy(pos_hbm.at[pl.ds(_N + b1, sub)], idx1b_v)
    la.wait()
    s0a = pltpu.async_copy(rows_a, xpad_hbm.at[idx0a_v], ssem)
    s1a = pltpu.async_copy(rows_a, xpad_hbm.at[idx1a_v], ssem)
    lb.wait()
    s0b = pltpu.async_copy(rows_b, xpad_hbm.at[idx0b_v], ssem)
    s1b = pltpu.async_copy(rows_b, xpad_hbm.at[idx1b_v], ssem)
    s0a.wait()
    s1a.wait()
    s0b.wait()
    s1b.wait()


def _shared_body(x_ref, sg_ref, su_ref, sd_ref, y_ref):
    x = x_ref[...].astype(jnp.bfloat16)
    sg = sg_ref[0].astype(jnp.bfloat16)
    su = su_ref[0].astype(jnp.bfloat16)
    sd = sd_ref[0].astype(jnp.bfloat16)
    g = lax.dot_general(x, sg, (((1,), (1,)), ((), ())),
                        preferred_element_type=jnp.float32)
    u = lax.dot_general(x, su, (((1,), (1,)), ((), ())),
                        preferred_element_type=jnp.float32)
    act = (g * jax.nn.sigmoid(g) * u).astype(jnp.bfloat16)
    y = lax.dot_general(act, sd, (((1,), (1,)), ((), ())),
                        preferred_element_type=jnp.float32)
    y_ref[...] = y.astype(jnp.bfloat16)


def _gmm_body(eid_ref, bid_ref, valid_ref, x_ref, wg_ref, wu_ref, wd_ref,
              y_ref):
    s = pl.program_id(0)

    @pl.when(valid_ref[s] == 1)
    def _():
        x = x_ref[...].astype(jnp.bfloat16)
        wg = wg_ref[0].astype(jnp.bfloat16)
        wu = wu_ref[0].astype(jnp.bfloat16)
        wd = wd_ref[0].astype(jnp.bfloat16)
        g = lax.dot_general(x, wg, (((1,), (1,)), ((), ())),
                            preferred_element_type=jnp.float32)
        u = lax.dot_general(x, wu, (((1,), (1,)), ((), ())),
                            preferred_element_type=jnp.float32)
        act = (g * jax.nn.sigmoid(g) * u).astype(jnp.bfloat16)
        y_ref[...] = lax.dot_general(act, wd, (((1,), (1,)), ((), ())),
                                     preferred_element_type=jnp.float32)


def _sc_gather_body(ypad_hbm, pos_hbm, y0g_hbm, y1g_hbm,
                    buf_a, buf_b, buf_c,
                    i0a_v, i1a_v, i0b_v, i1b_v,
                    sem_a, sem_b, sem_c, wsem_a, wsem_b, wsem_c):
    wid = lax.axis_index("s") * 2 + lax.axis_index("c")
    tpw = _N // _NW
    sub = tpw // 2
    b0 = wid * tpw
    b1 = b0 + sub
    pltpu.sync_copy(pos_hbm.at[pl.ds(b0, sub)], i0a_v)
    pltpu.sync_copy(pos_hbm.at[pl.ds(_N + b0, sub)], i1a_v)
    pltpu.sync_copy(pos_hbm.at[pl.ds(b1, sub)], i0b_v)
    pltpu.sync_copy(pos_hbm.at[pl.ds(_N + b1, sub)], i1b_v)
    g0 = pltpu.async_copy(ypad_hbm.at[i0a_v], buf_a, sem_a)
    g1 = pltpu.async_copy(ypad_hbm.at[i1a_v], buf_b, sem_b)
    g0.wait()
    w_a = pltpu.async_copy(buf_a, y0g_hbm.at[pl.ds(b0, sub)], wsem_a)
    g2 = pltpu.async_copy(ypad_hbm.at[i0b_v], buf_c, sem_c)
    g1.wait()
    w_b = pltpu.async_copy(buf_b, y1g_hbm.at[pl.ds(b0, sub)], wsem_b)
    w_a.wait()
    g3 = pltpu.async_copy(ypad_hbm.at[i1b_v], buf_a, sem_a)
    g2.wait()
    w_c = pltpu.async_copy(buf_c, y0g_hbm.at[pl.ds(b1, sub)], wsem_c)
    g3.wait()
    w_a2 = pltpu.async_copy(buf_a, y1g_hbm.at[pl.ds(b1, sub)], wsem_a)
    w_b.wait()
    w_c.wait()
    w_a2.wait()


def _add_body(y0_ref, y1_ref, ysh_ref, w1_ref, w2_ref, out_ref):
    w1 = w1_ref[...][:, :1]
    w2 = w2_ref[...][:, :1]
    ysh = ysh_ref[...].astype(jnp.float32)
    out_ref[...] = w1 * y0_ref[...] + w2 * y1_ref[...] + ysh


def kernel(hidden_states, gate_w, Wg, Wu, Wd, Sg, Su, Sd):
    b, t, d = hidden_states.shape
    n = b * t
    flat = hidden_states.reshape(n, d)
    gwt = jnp.pad(gate_w.T, ((0, 0), (0, 128 - _E)))

    pos, w1b, w2b, meta, aux = pl.pallas_call(
        _router_body,
        out_shape=(
            jax.ShapeDtypeStruct((2 * n, 1), jnp.int32),
            jax.ShapeDtypeStruct((n, 16), jnp.float32),
            jax.ShapeDtypeStruct((n, 16), jnp.float32),
            jax.ShapeDtypeStruct((8, 128), jnp.int32),
            jax.ShapeDtypeStruct((1, 1), jnp.float32),
        ),
        in_specs=[
            pl.BlockSpec((n, d), lambda: (0, 0)),
            pl.BlockSpec((d, 128), lambda: (0, 0)),
        ],
        out_specs=(
            pl.BlockSpec((2 * n, 1), lambda: (0, 0)),
            pl.BlockSpec((n, 16), lambda: (0, 0)),
            pl.BlockSpec((n, 16), lambda: (0, 0)),
            pl.BlockSpec((8, 128), lambda: (0, 0)),
            pl.BlockSpec(memory_space=pltpu.SMEM),
        ),
    )(flat, gwt)

    pos_all = pos.reshape(2 * n)
    eid = meta[0, :_G]
    bid = meta[1, :_G]
    blk_valid = meta[2, :_G]

    mesh = plsc.VectorSubcoreMesh(core_axis_name="c", subcore_axis_name="s")
    sc_scatter = functools.partial(
        pl.kernel, mesh=mesh,
        out_type=jax.ShapeDtypeStruct((_MR, d), jnp.float32),
        scratch_types=[
            pltpu.VMEM((32, d), jnp.float32),
            pltpu.VMEM((32, d), jnp.float32),
            pltpu.VMEM((32,), jnp.int32),
            pltpu.VMEM((32,), jnp.int32),
            pltpu.VMEM((32,), jnp.int32),
            pltpu.VMEM((32,), jnp.int32),
            pltpu.SemaphoreType.DMA,
            pltpu.SemaphoreType.DMA,
            pltpu.SemaphoreType.DMA,
        ],
    )(_sc_scatter_body)
    x_pad = sc_scatter(flat, pos_all)

    ysh = pl.pallas_call(
        _shared_body,
        grid=(4,),
        out_shape=jax.ShapeDtypeStruct((n, d), jnp.bfloat16),
        in_specs=[
            pl.BlockSpec((n // 4, d), lambda s: (s, 0)),
            pl.BlockSpec((1, _I, d), lambda s: (0, 0, 0)),
            pl.BlockSpec((1, _I, d), lambda s: (0, 0, 0)),
            pl.BlockSpec((1, d, _I), lambda s: (0, 0, 0)),
        ],
        out_specs=pl.BlockSpec((n // 4, d), lambda s: (s, 0)),
    )(flat, Sg, Su, Sd)

    grid_spec = pltpu.PrefetchScalarGridSpec(
        num_scalar_prefetch=3,
        grid=(_G,),
        in_specs=[
            pl.BlockSpec((_BM, d), lambda s, eid, bid, vld: (bid[s], 0)),
            pl.BlockSpec((1, _I, d), lambda s, eid, bid, vld: (eid[s], 0, 0)),
            pl.BlockSpec((1, _I, d), lambda s, eid, bid, vld: (eid[s], 0, 0)),
            pl.BlockSpec((1, d, _I), lambda s, eid, bid, vld: (eid[s], 0, 0)),
        ],
        out_specs=pl.BlockSpec((_BM, d), lambda s, eid, bid, vld: (bid[s], 0)),
    )
    y_pad = pl.pallas_call(
        _gmm_body,
        grid_spec=grid_spec,
        out_shape=jax.ShapeDtypeStruct((_MR, d), jnp.float32),
    )(eid, bid, blk_valid, x_pad, Wg, Wu, Wd)

    sc_gather = functools.partial(
        pl.kernel, mesh=mesh,
        out_type=(
            jax.ShapeDtypeStruct((n, d), jnp.float32),
            jax.ShapeDtypeStruct((n, d), jnp.float32),
        ),
        scratch_types=[
            pltpu.VMEM((32, d), jnp.float32),
            pltpu.VMEM((32, d), jnp.float32),
            pltpu.VMEM((32, d), jnp.float32),
            pltpu.VMEM((32,), jnp.int32),
            pltpu.VMEM((32,), jnp.int32),
            pltpu.VMEM((32,), jnp.int32),
            pltpu.VMEM((32,), jnp.int32),
            pltpu.SemaphoreType.DMA,
            pltpu.SemaphoreType.DMA,
            pltpu.SemaphoreType.DMA,
            pltpu.SemaphoreType.DMA,
            pltpu.SemaphoreType.DMA,
            pltpu.SemaphoreType.DMA,
        ],
    )(_sc_gather_body)
    y0g, y1g = sc_gather(y_pad, pos_all)

    out_flat = pl.pallas_call(
        _add_body,
        grid=(4,),
        out_shape=jax.ShapeDtypeStruct((n, d), jnp.float32),
        in_specs=[
            pl.BlockSpec((n // 4, d), lambda s: (s, 0)),
            pl.BlockSpec((n // 4, d), lambda s: (s, 0)),
            pl.BlockSpec((n // 4, d), lambda s: (s, 0)),
            pl.BlockSpec((n // 4, 16), lambda s: (s, 0)),
            pl.BlockSpec((n // 4, 16), lambda s: (s, 0)),
        ],
        out_specs=pl.BlockSpec((n // 4, d), lambda s: (s, 0)),
    )(y0g, y1g, ysh, w1b, w2b)

    return out_flat.reshape(b, t, d), aux[0, 0]
